# baseline (speedup 1.0000x reference)
import jax
import jax.numpy as jnp
from jax import lax
from jax.experimental import pallas as pl
from jax.experimental.pallas import tpu as pltpu
from jax.experimental.pallas import tpu_sc as plsc

N_RES = 100000
WORDS = N_RES * 12
GROUPS = WORDS // 48

_info = plsc.get_sparse_core_info()
NC = _info.num_cores
NS = _info.num_subcores
NW = NC * NS

G_MAIN = -(-GROUPS // NW)
G_LAST = GROUPS - G_MAIN * (NW - 1)
W_MAIN = G_MAIN * 48
W_LAST = G_LAST * 48
PAD = 8


def _body(x_hbm, out_hbm, buf, obuf):
    wid = lax.axis_index("s") * NC + lax.axis_index("c")
    base = wid * W_MAIN

    @pl.when(wid < NW - 1)
    def _():
        pltpu.sync_copy(x_hbm.at[pl.ds(base, W_MAIN)],
                        buf.at[pl.ds(PAD, W_MAIN)])

    @pl.when(wid == NW - 1)
    def _():
        pltpu.sync_copy(x_hbm.at[pl.ds(base, W_LAST)],
                        buf.at[pl.ds(PAD, W_LAST)])

    lanes = lax.iota(jnp.int32, 16)
    masks = []
    for j in range(3):
        col = jnp.remainder(lanes + (16 * j) % 12, 12)
        take_fwd = (col >= 3) & (col <= 5)
        take_bwd = (col >= 6) & (col <= 8)
        masks.append((take_fwd, take_bwd))

    ngroups = jnp.where(wid == NW - 1, G_LAST, G_MAIN)

    def step(k, carry):
        s = PAD + k * 48
        for j in range(3):
            off = s + 16 * j
            ident = buf[pl.ds(off, 16)]
            fwd = buf[pl.ds(off + 3, 16)]
            bwd = buf[pl.ds(off - 3, 16)]
            mf, mb = masks[j]
            out = jnp.where(mf, fwd, jnp.where(mb, bwd, ident))
            obuf[pl.ds(k * 48 + 16 * j, 16)] = out
        return carry

    lax.fori_loop(0, ngroups, step, 0)

    @pl.when(wid < NW - 1)
    def _():
        pltpu.sync_copy(obuf.at[pl.ds(0, W_MAIN)],
                        out_hbm.at[pl.ds(base, W_MAIN)])

    @pl.when(wid == NW - 1)
    def _():
        pltpu.sync_copy(obuf.at[pl.ds(0, W_LAST)],
                        out_hbm.at[pl.ds(base, W_LAST)])


def kernel(X):
    mesh = plsc.VectorSubcoreMesh(core_axis_name="c", subcore_axis_name="s")
    f = pl.kernel(
        _body,
        mesh=mesh,
        compiler_params=pltpu.CompilerParams(needs_layout_passes=False),
        out_type=jax.ShapeDtypeStruct((WORDS,), jnp.float32),
        scratch_types=[
            pltpu.VMEM((PAD + W_MAIN + PAD,), jnp.float32),
            pltpu.VMEM((W_MAIN,), jnp.float32),
        ],
    )
    return f(X.reshape(-1)).reshape(N_RES, 4, 3)


# speedup vs baseline: 1.0149x; 1.0149x over previous
import jax
import jax.numpy as jnp
from jax import lax
from jax.experimental import pallas as pl
from jax.experimental.pallas import tpu as pltpu
from jax.experimental.pallas import tpu_sc as plsc

N_RES = 100000
WORDS = N_RES * 12
GW = 192
GROUPS = WORDS // GW

_info = plsc.get_sparse_core_info()
NC = _info.num_cores
NS = _info.num_subcores
NW = NC * NS

G_MAIN = -(-GROUPS // NW)
G_LAST = GROUPS - G_MAIN * (NW - 1)
W_MAIN = G_MAIN * GW
W_LAST = G_LAST * GW


def _body(x_hbm, out_hbm, buf):
    wid = lax.axis_index("s") * NC + lax.axis_index("c")
    base = wid * W_MAIN

    @pl.when(wid < NW - 1)
    def _():
        pltpu.sync_copy(x_hbm.at[pl.ds(base, W_MAIN)], buf.at[pl.ds(0, W_MAIN)])

    @pl.when(wid == NW - 1)
    def _():
        pltpu.sync_copy(x_hbm.at[pl.ds(base, W_LAST)], buf.at[pl.ds(0, W_LAST)])

    lanes = lax.iota(jnp.int32, 16)
    pats = []
    for j in range(3):
        p = lanes + 16 * j
        res = p // 3
        c = p - 3 * res
        pa = 12 * res + 3 + c
        pats.append((pa, pa + 3))

    ngroups = jnp.where(wid == NW - 1, G_LAST, G_MAIN)

    def step(k, carry):
        s = k * GW
        for j in range(3):
            ia = s + pats[j][0]
            ib = s + pats[j][1]
            v1 = plsc.load_gather(buf, [ia])
            v2 = plsc.load_gather(buf, [ib])
            plsc.store_scatter(buf, [ia], v2)
            plsc.store_scatter(buf, [ib], v1)
        return carry

    lax.fori_loop(0, ngroups, step, 0)

    @pl.when(wid < NW - 1)
    def _():
        pltpu.sync_copy(buf.at[pl.ds(0, W_MAIN)], out_hbm.at[pl.ds(base, W_MAIN)])

    @pl.when(wid == NW - 1)
    def _():
        pltpu.sync_copy(buf.at[pl.ds(0, W_LAST)], out_hbm.at[pl.ds(base, W_LAST)])


def kernel(X):
    mesh = plsc.VectorSubcoreMesh(core_axis_name="c", subcore_axis_name="s")
    f = pl.kernel(
        _body,
        mesh=mesh,
        compiler_params=pltpu.CompilerParams(needs_layout_passes=False),
        out_type=jax.ShapeDtypeStruct((WORDS,), jnp.float32),
        scratch_types=[
            pltpu.VMEM((W_MAIN,), jnp.float32),
        ],
    )
    return f(X.reshape(-1)).reshape(N_RES, 4, 3)


# speedup vs baseline: 1.2748x; 1.2561x over previous
import jax
import jax.numpy as jnp
from jax import lax
from jax.experimental import pallas as pl
from jax.experimental.pallas import tpu as pltpu
from jax.experimental.pallas import tpu_sc as plsc

N_RES = 100000
PLANE = N_RES * 3
WORDS = PLANE * 4

_info = plsc.get_sparse_core_info()
NC = _info.num_cores
NS = _info.num_subcores
NW = NC * NS
KPP = NW // 4

CH = 37504
CH_LAST = PLANE - 7 * CH


def _body(x_hbm, out_hbm, buf):
    wid = lax.axis_index("s") * NC + lax.axis_index("c")
    p = wid // KPP
    k = wid - p * KPP
    sp = jnp.where(p == 1, 2, jnp.where(p == 2, 1, p))
    src = sp * PLANE + k * CH
    dst = p * PLANE + k * CH

    @pl.when(k < KPP - 1)
    def _():
        pltpu.sync_copy(x_hbm.at[pl.ds(src, CH)], buf.at[pl.ds(0, CH)])
        pltpu.sync_copy(buf.at[pl.ds(0, CH)], out_hbm.at[pl.ds(dst, CH)])

    @pl.when(k == KPP - 1)
    def _():
        pltpu.sync_copy(x_hbm.at[pl.ds(src, CH_LAST)], buf.at[pl.ds(0, CH_LAST)])
        pltpu.sync_copy(buf.at[pl.ds(0, CH_LAST)], out_hbm.at[pl.ds(dst, CH_LAST)])


def kernel(X):
    mesh = plsc.VectorSubcoreMesh(core_axis_name="c", subcore_axis_name="s")
    f = pl.kernel(
        _body,
        mesh=mesh,
        out_type=jax.ShapeDtypeStruct((WORDS,), jnp.float32),
        scratch_types=[
            pltpu.VMEM((CH,), jnp.float32),
        ],
    )
    flat = jnp.transpose(X, (1, 0, 2)).reshape(-1)
    y = f(flat)
    return jnp.transpose(y.reshape(4, N_RES, 3), (1, 0, 2))
